# baseline (device time: 1127683 ns/iter reference)
import jax
import jax.numpy as jnp
from jax import lax
from jax.experimental import pallas as pl
from jax.experimental.pallas import tpu as pltpu

N = 32


def kernel(x, Win0, Wout0, Win1, Wout1, Win2, Wout2):
    m_per, d = x.shape
    M = N * m_per

    def body(x_ref, win0, wout0, win1, wout1, win2, wout2,
             out_ref, xfull, acc, comm, send_sems, recv_sems, credit_sem):
        my = lax.axis_index("i")
        left = (my - 1) % N
        right = (my + 1) % N

        barrier_sem = pltpu.get_barrier_semaphore()
        for nbr in (left, right):
            pl.semaphore_signal(barrier_sem, inc=1, device_id=(nbr,),
                                device_id_type=pl.DeviceIdType.MESH)
        pl.semaphore_wait(barrier_sem, 2)

        def chunk(ref, idx):
            return ref.at[pl.ds(idx * m_per, m_per)]

        def ring_hop(h, src_slice, dst_slice, consume):
            if h >= 2:
                pl.semaphore_wait(credit_sem, 1)
            rdma = pltpu.make_async_remote_copy(
                src_ref=src_slice,
                dst_ref=dst_slice,
                send_sem=send_sems.at[h % 2],
                recv_sem=recv_sems.at[h % 2],
                device_id=(right,),
                device_id_type=pl.DeviceIdType.MESH,
            )
            rdma.start()
            rdma.wait()
            consume()
            pl.semaphore_signal(credit_sem, inc=1, device_id=(left,),
                                device_id_type=pl.DeviceIdType.MESH)

        def drain_credits():
            pl.semaphore_wait(credit_sem, 2)

        xfull[pl.ds(my * m_per, m_per), :] = x_ref[...]
        for h in range(N - 1):
            send_idx = (my - h) % N
            ring_hop(h, chunk(xfull, send_idx), chunk(xfull, send_idx),
                     lambda: None)
        drain_credits()

        layers = [
            (win0, wout0, xfull, acc),
            (win1, wout1, acc, xfull),
            (win2, wout2, xfull, out_ref),
        ]
        NB = 8
        bs = M // NB
        for win, wout, src, dst in layers:
            wv = win[...]
            wo = wout[...]
            for b in range(NB):
                xb = src[pl.ds(b * bs, bs), :]
                hb = jnp.maximum(
                    jnp.dot(xb, wv, preferred_element_type=jnp.float32), 0.0)
                dst[pl.ds(b * bs, bs), :] = jnp.dot(
                    hb, wo, preferred_element_type=jnp.float32)

            for s in range(N - 1):
                cs = (my - s) % N
                cr = (my - s - 1) % N

                def rs_consume(cr=cr, s=s, dst=dst):
                    dst[pl.ds(cr * m_per, m_per), :] = (
                        dst[pl.ds(cr * m_per, m_per), :] + comm[s % 2])

                ring_hop(s, chunk(dst, cs), comm.at[s % 2], rs_consume)
            drain_credits()

            for g in range(N - 1):
                si = (my + 1 - g) % N
                ring_hop(g, chunk(dst, si), chunk(dst, si), lambda: None)
            drain_credits()

    return pl.pallas_call(
        body,
        out_shape=jax.ShapeDtypeStruct((M, d), jnp.float32),
        in_specs=[pl.BlockSpec(memory_space=pltpu.VMEM)] * 7,
        out_specs=pl.BlockSpec(memory_space=pltpu.VMEM),
        scratch_shapes=[
            pltpu.VMEM((M, d), jnp.float32),
            pltpu.VMEM((M, d), jnp.float32),
            pltpu.VMEM((2, m_per, d), jnp.float32),
            pltpu.SemaphoreType.DMA((2,)),
            pltpu.SemaphoreType.DMA((2,)),
            pltpu.SemaphoreType.REGULAR,
        ],
        compiler_params=pltpu.CompilerParams(collective_id=0),
    )(x, Win0, Wout0, Win1, Wout1, Win2, Wout2)


# device time: 808924 ns/iter; 1.3941x vs baseline; 1.3941x over previous
import jax
import jax.numpy as jnp
from jax import lax
from jax.experimental import pallas as pl
from jax.experimental.pallas import tpu as pltpu

N = 32
R_HOPS = 16
L_HOPS = 15


def kernel(x, Win0, Wout0, Win1, Wout1, Win2, Wout2):
    m_per, d = x.shape
    M = N * m_per

    def body(x_ref, win0, wout0, win1, wout1, win2, wout2,
             out_ref, buf0, buf1, commR, commL,
             ssR, rsR, ssL, rsL, creditR, creditL):
        my = lax.axis_index("i")
        left = (my - 1) % N
        right = (my + 1) % N

        barrier_sem = pltpu.get_barrier_semaphore()
        for nbr in (left, right):
            pl.semaphore_signal(barrier_sem, inc=1, device_id=(nbr,),
                                device_id_type=pl.DeviceIdType.MESH)
        pl.semaphore_wait(barrier_sem, 2)

        def chunk(ref, idx):
            return ref.at[pl.ds((idx % N) * m_per, m_per)]

        def copy_r(src, dst, slot):
            return pltpu.make_async_remote_copy(
                src_ref=src, dst_ref=dst,
                send_sem=ssR.at[slot], recv_sem=rsR.at[slot],
                device_id=(right,), device_id_type=pl.DeviceIdType.MESH)

        def copy_l(src, dst, slot):
            return pltpu.make_async_remote_copy(
                src_ref=src, dst_ref=dst,
                send_sem=ssL.at[slot], recv_sem=rsL.at[slot],
                device_id=(left,), device_id_type=pl.DeviceIdType.MESH)

        def signal(sem, dev):
            pl.semaphore_signal(sem, inc=1, device_id=(dev,),
                                device_id_type=pl.DeviceIdType.MESH)

        def drain_credits():
            pl.semaphore_wait(creditR, 2)
            pl.semaphore_wait(creditL, 2)

        def ag_phase(buf, on_chunk):
            def start_hop(h):
                if h >= 2:
                    pl.semaphore_wait(creditR, 1)
                if 2 <= h < L_HOPS:
                    pl.semaphore_wait(creditL, 1)
                rR = copy_r(chunk(buf, my - h), chunk(buf, my - h), h % 2)
                rR.start()
                rL = None
                if h < L_HOPS:
                    rL = copy_l(chunk(buf, my + h), chunk(buf, my + h), h % 2)
                    rL.start()
                return rR, rL

            cur = start_hop(0)
            if on_chunk is not None:
                on_chunk(my)
            for h in range(R_HOPS):
                rR, rL = cur
                rR.wait()
                if rL is not None:
                    rL.wait()
                signal(creditR, left)
                if rL is not None:
                    signal(creditL, right)
                cur = start_hop(h + 1) if h + 1 < R_HOPS else None
                if on_chunk is not None:
                    on_chunk(my - 1 - h)
                    if rL is not None:
                        on_chunk(my + 1 + h)
            drain_credits()

        def rs_phase(dst):
            def start_hop(s):
                if s >= 2:
                    pl.semaphore_wait(creditR, 1)
                if 2 <= s < L_HOPS:
                    pl.semaphore_wait(creditL, 1)
                rR = copy_r(chunk(dst, my + R_HOPS - s), commR.at[s % 2],
                            s % 2)
                rR.start()
                rL = None
                if s < L_HOPS:
                    rL = copy_l(chunk(dst, my - L_HOPS + s), commL.at[s % 2],
                                s % 2)
                    rL.start()
                return rR, rL

            cur = start_hop(0)
            for s in range(R_HOPS):
                rR, rL = cur
                rR.wait()
                if rL is not None:
                    rL.wait()
                iR = my + R_HOPS - 1 - s
                cR = chunk(dst, iR)
                cR[...] = cR[...] + commR[s % 2]
                signal(creditR, left)
                if rL is not None:
                    iL = my - L_HOPS + 1 + s
                    cL = chunk(dst, iL)
                    cL[...] = cL[...] + commL[s % 2]
                    signal(creditL, right)
                cur = start_hop(s + 1) if s + 1 < R_HOPS else None
            drain_credits()

        def make_on_chunk(src, win, wout, dstbuf):
            wv = win[...]
            wo = wout[...]

            def on_chunk(c):
                xb = chunk(src, c)[...]
                hb = jnp.maximum(
                    jnp.dot(xb, wv, preferred_element_type=jnp.float32), 0.0)
                chunk(dstbuf, c)[...] = jnp.dot(
                    hb, wo, preferred_element_type=jnp.float32)
            return on_chunk

        buf0[pl.ds(my * m_per, m_per), :] = x_ref[...]
        ag_phase(buf0, make_on_chunk(buf0, win0, wout0, buf1))
        rs_phase(buf1)
        ag_phase(buf1, make_on_chunk(buf1, win1, wout1, buf0))
        rs_phase(buf0)
        ag_phase(buf0, make_on_chunk(buf0, win2, wout2, out_ref))
        rs_phase(out_ref)
        ag_phase(out_ref, None)

    return pl.pallas_call(
        body,
        out_shape=jax.ShapeDtypeStruct((M, d), jnp.float32),
        in_specs=[pl.BlockSpec(memory_space=pltpu.VMEM)] * 7,
        out_specs=pl.BlockSpec(memory_space=pltpu.VMEM),
        scratch_shapes=[
            pltpu.VMEM((M, d), jnp.float32),
            pltpu.VMEM((M, d), jnp.float32),
            pltpu.VMEM((2, m_per, d), jnp.float32),
            pltpu.VMEM((2, m_per, d), jnp.float32),
            pltpu.SemaphoreType.DMA((2,)),
            pltpu.SemaphoreType.DMA((2,)),
            pltpu.SemaphoreType.DMA((2,)),
            pltpu.SemaphoreType.DMA((2,)),
            pltpu.SemaphoreType.REGULAR,
            pltpu.SemaphoreType.REGULAR,
        ],
        compiler_params=pltpu.CompilerParams(collective_id=0),
    )(x, Win0, Wout0, Win1, Wout1, Win2, Wout2)


# device time: 235317 ns/iter; 4.7922x vs baseline; 3.4376x over previous
import jax
import jax.numpy as jnp
from jax import lax
from jax.experimental import pallas as pl
from jax.experimental.pallas import tpu as pltpu

N = 32
R_HOPS = 16
L_HOPS = 15
K_SUB = 2

PERM = [0, 8, 16, 24, 27, 19, 11, 12, 20, 28, 31, 23, 15, 7, 4, 3,
        2, 5, 6, 14, 22, 30, 29, 21, 13, 10, 18, 26, 25, 17, 9, 1]
INV = [0, 31, 16, 15, 14, 17, 18, 13, 1, 30, 25, 6, 7, 24, 19, 12,
       2, 29, 26, 5, 8, 23, 20, 11, 3, 28, 27, 4, 9, 22, 21, 10]

STREAMS = ("A", "S")
DIRS = ("R", "L")


def kernel(x, Win0, Wout0, Win1, Wout1, Win2, Wout2):
    m_per, d = x.shape
    M = N * m_per
    m_half = m_per // K_SUB

    n_dma = 2 * len(STREAMS) * len(DIRS) * K_SUB
    n_cred = len(STREAMS) * len(DIRS) * K_SUB

    def body(x_ref, win0, wout0, win1, wout1, win2, wout2,
             out_ref, *scratch):
        buf0, buf1, commR, commL, perm_ref, inv_ref = scratch[:6]
        sems = scratch[6:6 + n_dma]
        creds = scratch[6 + n_dma:6 + n_dma + n_cred]
        phase_sem = scratch[6 + n_dma + n_cred]

        my = lax.axis_index("i")
        for k in range(N):
            perm_ref[k] = PERM[k]
            inv_ref[k] = INV[k]
        rp = inv_ref[my]
        left = perm_ref[(rp - 1) % N]
        right = perm_ref[(rp + 1) % N]

        send_sems, recv_sems, credits = {}, {}, {}
        i = 0
        j = 0
        for st in STREAMS:
            for dirn in DIRS:
                for t in range(K_SUB):
                    send_sems[(st, dirn, t)] = sems[i]
                    recv_sems[(st, dirn, t)] = sems[i + 1]
                    i += 2
                    credits[(st, dirn, t)] = creds[j]
                    j += 1
        peer = {"R": (right, left), "L": (left, right)}

        barrier_sem = pltpu.get_barrier_semaphore()
        for nbr in (left, right):
            pl.semaphore_signal(barrier_sem, inc=1, device_id=(nbr,),
                                device_id_type=pl.DeviceIdType.MESH)
        pl.semaphore_wait(barrier_sem, 2)

        def half(ref, ridx, t):
            return ref.at[pl.ds(perm_ref[ridx % N] * m_per + t * m_half,
                                m_half)]

        def chunk_blk(ridx):
            return pl.ds(perm_ref[ridx % N] * m_per, m_per)

        def start(key, src, dst, slot):
            rdma = pltpu.make_async_remote_copy(
                src_ref=src, dst_ref=dst,
                send_sem=send_sems[key].at[slot],
                recv_sem=recv_sems[key].at[slot],
                device_id=(peer[key[1]][0],),
                device_id_type=pl.DeviceIdType.MESH)
            rdma.start()
            return rdma

        def wait_credit(key):
            pl.semaphore_wait(credits[key], 1)

        def send_credit(key):
            pl.semaphore_signal(credits[key], inc=1,
                                device_id=(peer[key[1]][1],),
                                device_id_type=pl.DeviceIdType.MESH)

        def drain_credits(streams):
            for key, sem in credits.items():
                if key[0] in streams:
                    pl.semaphore_wait(sem, 2)

        def add_half(P, ridx, t, comm, slot):
            hh = half(P, ridx, t)
            hh[...] = (hh[...].astype(jnp.float32) +
                       comm[slot, t].astype(jnp.float32)).astype(jnp.bfloat16)

        def fused_phase(I, P, b, on_chunk):

            def ag_start(h):
                cur = {}
                for t in range(K_SUB):
                    if h >= 2:
                        wait_credit(("A", "R", t))
                    s = half(I, b - h, t)
                    cur[("R", t)] = start(("A", "R", t), s, s, h % 2)
                    if h < L_HOPS:
                        if h >= 2:
                            wait_credit(("A", "L", t))
                        s = half(I, b + h, t)
                        cur[("L", t)] = start(("A", "L", t), s, s, h % 2)
                return cur

            def rs_start(s):
                cur = {}
                for t in range(K_SUB):
                    if s >= 2:
                        wait_credit(("S", "R", t))
                    cur[("R", t)] = start(("S", "R", t),
                                          half(P, b - s, t),
                                          commR.at[s % 2, t], s % 2)
                    if s < L_HOPS:
                        if s >= 2:
                            wait_credit(("S", "L", t))
                        cur[("L", t)] = start(("S", "L", t),
                                              half(P, b + 1 + s, t),
                                              commL.at[s % 2, t], s % 2)
                return cur

            ag_cur = ag_start(0)
            on_chunk(b)
            rs_cur = {}
            n_iter = R_HOPS + 1 if P is not None else R_HOPS
            for k in range(n_iter):
                if k <= R_HOPS - 1:
                    for t in range(K_SUB):
                        ag_cur[("R", t)].wait()
                        send_credit(("A", "R", t))
                        if ("L", t) in ag_cur:
                            ag_cur[("L", t)].wait()
                            send_credit(("A", "L", t))
                ag_nxt = ag_start(k + 1) if k + 1 <= R_HOPS - 1 else {}
                if k <= R_HOPS - 1:
                    on_chunk(b - 1 - k)
                    if k <= L_HOPS - 1:
                        on_chunk(b + 1 + k)
                if P is not None and k >= 1:
                    s = k - 1
                    for t in range(K_SUB):
                        rs_cur[("R", t)].wait()
                        add_half(P, b - 1 - s, t, commR, s % 2)
                        send_credit(("S", "R", t))
                        if ("L", t) in rs_cur:
                            rs_cur[("L", t)].wait()
                            add_half(P, b + 2 + s, t, commL, s % 2)
                            send_credit(("S", "L", t))
                if P is not None and k <= R_HOPS - 1:
                    rs_cur = rs_start(k)
                ag_cur = ag_nxt
            drain_credits(("A", "S") if P is not None else ("A",))
            if P is not None:
                for nbr in (left, right):
                    pl.semaphore_signal(phase_sem, inc=1, device_id=(nbr,),
                                        device_id_type=pl.DeviceIdType.MESH)
                pl.semaphore_wait(phase_sem, 2)

        def make_on_chunk(src, win, wout, dstbuf):
            wv = win[...]
            wo = wout[...]

            def on_chunk(c):
                blk = chunk_blk(c)
                xb = src[blk, :]
                hb = jnp.maximum(
                    jnp.dot(xb, wv, preferred_element_type=jnp.float32), 0.0)
                dstbuf[blk, :] = jnp.dot(
                    hb, wo,
                    preferred_element_type=jnp.float32).astype(jnp.bfloat16)
            return on_chunk

        buf0[pl.ds(my * m_per, m_per), :] = x_ref[...].astype(jnp.bfloat16)
        fused_phase(buf0, buf1, rp,
                    make_on_chunk(buf0, win0, wout0, buf1))
        fused_phase(buf1, buf0, rp + R_HOPS,
                    make_on_chunk(buf1, win1, wout1, buf0))
        fused_phase(buf0, buf1, rp,
                    make_on_chunk(buf0, win2, wout2, buf1))

        def emit(c):
            blk = chunk_blk(c)
            out_ref[blk, :] = buf1[blk, :].astype(jnp.float32)
        fused_phase(buf1, None, rp + R_HOPS, emit)

    return pl.pallas_call(
        body,
        out_shape=jax.ShapeDtypeStruct((M, d), jnp.float32),
        in_specs=[pl.BlockSpec(memory_space=pltpu.VMEM)] * 7,
        out_specs=pl.BlockSpec(memory_space=pltpu.VMEM),
        scratch_shapes=(
            [
                pltpu.VMEM((M, d), jnp.bfloat16),
                pltpu.VMEM((M, d), jnp.bfloat16),
                pltpu.VMEM((2, K_SUB, m_half, d), jnp.bfloat16),
                pltpu.VMEM((2, K_SUB, m_half, d), jnp.bfloat16),
                pltpu.SMEM((N,), jnp.int32),
                pltpu.SMEM((N,), jnp.int32),
            ]
            + [pltpu.SemaphoreType.DMA((2,))] * n_dma
            + [pltpu.SemaphoreType.REGULAR] * n_cred
            + [pltpu.SemaphoreType.REGULAR]
        ),
        compiler_params=pltpu.CompilerParams(collective_id=0),
    )(x, Win0, Wout0, Win1, Wout1, Win2, Wout2)
